# Initial kernel scaffold; baseline (speedup 1.0000x reference)
import jax
import jax.numpy as jnp
from jax.experimental import pallas as pl


def kernel(src_hashes, pos_dst_hashes, neg_dst_hashes, W_src, W_tgt):
    raise NotImplementedError("write your pallas kernel here")


# speedup vs baseline: 2.4915x; 2.4915x over previous
import functools

import jax
import jax.numpy as jnp
from jax import lax
from jax.experimental import pallas as pl
from jax.experimental.pallas import tpu as pltpu
from jax.experimental.pallas import tpu_sc as plsc

TOTAL_BUCKETS = 16 * (2 ** 16)
EMB_DIM = 32
B = 16384
NUM_NEG = 8
M = 16

NC = 2
NS = 16
NW = NC * NS
L = 16

ROWS_PER_DMA = 128
POOLS_PER_DMA = ROWS_PER_DMA // M
IDX_BLOCK = 64
POOLS_PER_SUPER = IDX_BLOCK * POOLS_PER_DMA


def _pool_superchunk(table, idx2d, out_hbm, idx_row_base, out_row_base,
                     idx_v, rows_v, stage_v, sem):
    pltpu.sync_copy(idx2d.at[pl.ds(idx_row_base, IDX_BLOCK)], idx_v)

    def body(j, _):
        pltpu.async_copy(table.at[idx_v.at[j]], rows_v, sem).wait()
        for g in range(POOLS_PER_DMA):
            acc0 = jnp.zeros((L,), jnp.float32)
            acc1 = jnp.zeros((L,), jnp.float32)
            for h in range(M):
                r = g * M + h
                acc0 = acc0 + rows_v[r, 0:L]
                acc1 = acc1 + rows_v[r, L:EMB_DIM]
            row = j * POOLS_PER_DMA + g
            stage_v[row, 0:L] = acc0 * (1.0 / M)
            stage_v[row, L:EMB_DIM] = acc1 * (1.0 / M)
        return 0

    lax.fori_loop(0, IDX_BLOCK, body, 0)
    pltpu.sync_copy(stage_v, out_hbm.at[pl.ds(out_row_base, POOLS_PER_SUPER)])


def _sc_body(src_idx, pos_idx, neg_idx, w_src, w_tgt,
             su_out, tp_out, tn_out,
             idx_v, rows_v, stage_v, sem):
    wid = lax.axis_index("s") * NC + lax.axis_index("c")

    _pool_superchunk(w_src, src_idx, su_out, wid * IDX_BLOCK,
                     wid * POOLS_PER_SUPER, idx_v, rows_v, stage_v, sem)
    _pool_superchunk(w_tgt, pos_idx, tp_out, wid * IDX_BLOCK,
                     wid * POOLS_PER_SUPER, idx_v, rows_v, stage_v, sem)

    def neg_body(s, _):
        _pool_superchunk(w_tgt, neg_idx, tn_out,
                         wid * (IDX_BLOCK * NUM_NEG) + s * IDX_BLOCK,
                         wid * (POOLS_PER_SUPER * NUM_NEG) + s * POOLS_PER_SUPER,
                         idx_v, rows_v, stage_v, sem)
        return 0

    lax.fori_loop(0, NUM_NEG, neg_body, 0)


def _make_sc_pool():
    mesh = plsc.VectorSubcoreMesh(core_axis_name="c", subcore_axis_name="s",
                                  num_cores=NC, num_subcores=NS)
    return pl.kernel(
        _sc_body,
        out_type=[
            jax.ShapeDtypeStruct((B, EMB_DIM), jnp.float32),
            jax.ShapeDtypeStruct((B, EMB_DIM), jnp.float32),
            jax.ShapeDtypeStruct((B * NUM_NEG, EMB_DIM), jnp.float32),
        ],
        mesh=mesh,
        scratch_types=[
            pltpu.VMEM((IDX_BLOCK, ROWS_PER_DMA), jnp.int32),
            pltpu.VMEM((ROWS_PER_DMA, EMB_DIM), jnp.float32),
            pltpu.VMEM((POOLS_PER_SUPER, EMB_DIM), jnp.float32),
            pltpu.SemaphoreType.DMA,
        ],
        compiler_params=pltpu.CompilerParams(use_tc_tiling_on_sc=False),
    )


def _softplus(x):
    return jnp.maximum(x, 0.0) + jnp.log1p(jnp.exp(-jnp.abs(x)))


def _loss_body(su_ref, tp_ref, tn_ref, out_ref):
    su = su_ref[...]
    tp = tp_ref[...]
    acc = _softplus(-jnp.sum(su * tp, axis=1))
    for n in range(NUM_NEG):
        tn = tn_ref[:, n * EMB_DIM:(n + 1) * EMB_DIM]
        acc = acc + _softplus(jnp.sum(su * tn, axis=1))
    tot = jnp.sum(acc).reshape(1, 1)

    @pl.when(pl.program_id(0) == 0)
    def _():
        out_ref[...] = jnp.zeros((1, 1), jnp.float32)

    out_ref[...] += tot


_TC_BLOCK = 512


def _make_tc_loss():
    grid = (B // _TC_BLOCK,)
    return pl.pallas_call(
        _loss_body,
        grid=grid,
        in_specs=[
            pl.BlockSpec((_TC_BLOCK, EMB_DIM), lambda i: (i, 0)),
            pl.BlockSpec((_TC_BLOCK, EMB_DIM), lambda i: (i, 0)),
            pl.BlockSpec((_TC_BLOCK, EMB_DIM * NUM_NEG), lambda i: (i, 0)),
        ],
        out_specs=pl.BlockSpec((1, 1), lambda i: (0, 0)),
        out_shape=jax.ShapeDtypeStruct((1, 1), jnp.float32),
    )


@jax.jit
def kernel(src_hashes, pos_dst_hashes, neg_dst_hashes, W_src, W_tgt):
    src_idx = src_hashes.astype(jnp.int32).reshape(B * M // ROWS_PER_DMA,
                                                   ROWS_PER_DMA)
    pos_idx = pos_dst_hashes.astype(jnp.int32).reshape(B * M // ROWS_PER_DMA,
                                                       ROWS_PER_DMA)
    neg_idx = neg_dst_hashes.astype(jnp.int32).reshape(
        B * NUM_NEG * M // ROWS_PER_DMA, ROWS_PER_DMA)

    su, tp, tn = _make_sc_pool()(src_idx, pos_idx, neg_idx, W_src, W_tgt)
    tn2 = tn.reshape(B, NUM_NEG * EMB_DIM)
    tot = _make_tc_loss()(su, tp, tn2)
    return tot[0, 0] / B


# speedup vs baseline: 3.4011x; 1.3651x over previous
import functools

import jax
import jax.numpy as jnp
from jax import lax
from jax.experimental import pallas as pl
from jax.experimental.pallas import tpu as pltpu
from jax.experimental.pallas import tpu_sc as plsc

TOTAL_BUCKETS = 16 * (2 ** 16)
EMB_DIM = 32
B = 16384
NUM_NEG = 8
M = 16

NC = 2
NS = 16
NW = NC * NS
L = 16

ROWS_PER_DMA = 128
POOLS_PER_DMA = ROWS_PER_DMA // M
IDX_BLOCK = 64
POOLS_PER_SUPER = IDX_BLOCK * POOLS_PER_DMA


NBUF = 4


def _pool_superchunk(table, idx2d, out_hbm, idx_row_base, out_row_base,
                     idx_v, rows_v, stage_v, sems):
    pltpu.sync_copy(idx2d.at[pl.ds(idx_row_base, IDX_BLOCK)], idx_v)

    for b in range(NBUF):
        pltpu.async_copy(table.at[idx_v.at[b]], rows_v.at[b], sems[b])

    def outer(jj, _):
        for b in range(NBUF):
            j = jj * NBUF + b
            pltpu.make_async_copy(table.at[idx_v.at[j]], rows_v.at[b],
                                  sems[b]).wait()
            for g in range(POOLS_PER_DMA):
                acc0 = jnp.zeros((L,), jnp.float32)
                acc1 = jnp.zeros((L,), jnp.float32)
                for h in range(M):
                    r = g * M + h
                    acc0 = acc0 + rows_v[b, r, 0:L]
                    acc1 = acc1 + rows_v[b, r, L:EMB_DIM]
                row = j * POOLS_PER_DMA + g
                stage_v[row, 0:L] = acc0 * (1.0 / M)
                stage_v[row, L:EMB_DIM] = acc1 * (1.0 / M)

            @pl.when(j + NBUF < IDX_BLOCK)
            def _():
                pltpu.async_copy(table.at[idx_v.at[j + NBUF]], rows_v.at[b],
                                 sems[b])
        return 0

    lax.fori_loop(0, IDX_BLOCK // NBUF, outer, 0)
    pltpu.sync_copy(stage_v, out_hbm.at[pl.ds(out_row_base, POOLS_PER_SUPER)])


def _sc_body(src_idx, pos_idx, neg_idx, w_src, w_tgt,
             su_out, tp_out, tn_out,
             idx_v, rows_v, stage_v, sem0, sem1, sem2, sem3):
    wid = lax.axis_index("s") * NC + lax.axis_index("c")
    sems = (sem0, sem1, sem2, sem3)

    _pool_superchunk(w_src, src_idx, su_out, wid * IDX_BLOCK,
                     wid * POOLS_PER_SUPER, idx_v, rows_v, stage_v, sems)
    _pool_superchunk(w_tgt, pos_idx, tp_out, wid * IDX_BLOCK,
                     wid * POOLS_PER_SUPER, idx_v, rows_v, stage_v, sems)

    def neg_body(s, _):
        _pool_superchunk(w_tgt, neg_idx, tn_out,
                         wid * (IDX_BLOCK * NUM_NEG) + s * IDX_BLOCK,
                         wid * (POOLS_PER_SUPER * NUM_NEG) + s * POOLS_PER_SUPER,
                         idx_v, rows_v, stage_v, sems)
        return 0

    lax.fori_loop(0, NUM_NEG, neg_body, 0)


def _make_sc_pool():
    mesh = plsc.VectorSubcoreMesh(core_axis_name="c", subcore_axis_name="s",
                                  num_cores=NC, num_subcores=NS)
    return pl.kernel(
        _sc_body,
        out_type=[
            jax.ShapeDtypeStruct((B, EMB_DIM), jnp.float32),
            jax.ShapeDtypeStruct((B, EMB_DIM), jnp.float32),
            jax.ShapeDtypeStruct((B * NUM_NEG, EMB_DIM), jnp.float32),
        ],
        mesh=mesh,
        scratch_types=[
            pltpu.VMEM((IDX_BLOCK, ROWS_PER_DMA), jnp.int32),
            pltpu.VMEM((NBUF, ROWS_PER_DMA, EMB_DIM), jnp.float32),
            pltpu.VMEM((POOLS_PER_SUPER, EMB_DIM), jnp.float32),
            pltpu.SemaphoreType.DMA,
            pltpu.SemaphoreType.DMA,
            pltpu.SemaphoreType.DMA,
            pltpu.SemaphoreType.DMA,
        ],
        compiler_params=pltpu.CompilerParams(use_tc_tiling_on_sc=False),
    )


def _softplus(x):
    return jnp.maximum(x, 0.0) + jnp.log1p(jnp.exp(-jnp.abs(x)))


def _loss_body(su_ref, tp_ref, tn_ref, out_ref):
    su = su_ref[...]
    tp = tp_ref[...]
    acc = _softplus(-jnp.sum(su * tp, axis=1))
    for n in range(NUM_NEG):
        tn = tn_ref[:, n * EMB_DIM:(n + 1) * EMB_DIM]
        acc = acc + _softplus(jnp.sum(su * tn, axis=1))
    tot = jnp.sum(acc).reshape(1, 1)

    @pl.when(pl.program_id(0) == 0)
    def _():
        out_ref[...] = jnp.zeros((1, 1), jnp.float32)

    out_ref[...] += tot


_TC_BLOCK = 512


def _make_tc_loss():
    grid = (B // _TC_BLOCK,)
    return pl.pallas_call(
        _loss_body,
        grid=grid,
        in_specs=[
            pl.BlockSpec((_TC_BLOCK, EMB_DIM), lambda i: (i, 0)),
            pl.BlockSpec((_TC_BLOCK, EMB_DIM), lambda i: (i, 0)),
            pl.BlockSpec((_TC_BLOCK, EMB_DIM * NUM_NEG), lambda i: (i, 0)),
        ],
        out_specs=pl.BlockSpec((1, 1), lambda i: (0, 0)),
        out_shape=jax.ShapeDtypeStruct((1, 1), jnp.float32),
    )


@jax.jit
def kernel(src_hashes, pos_dst_hashes, neg_dst_hashes, W_src, W_tgt):
    src_idx = src_hashes.astype(jnp.int32).reshape(B * M // ROWS_PER_DMA,
                                                   ROWS_PER_DMA)
    pos_idx = pos_dst_hashes.astype(jnp.int32).reshape(B * M // ROWS_PER_DMA,
                                                       ROWS_PER_DMA)
    neg_idx = neg_dst_hashes.astype(jnp.int32).reshape(
        B * NUM_NEG * M // ROWS_PER_DMA, ROWS_PER_DMA)

    su, tp, tn = _make_sc_pool()(src_idx, pos_idx, neg_idx, W_src, W_tgt)
    tn2 = tn.reshape(B, NUM_NEG * EMB_DIM)
    tot = _make_tc_loss()(su, tp, tn2)
    return tot[0, 0] / B
